# baseline (device time: 86024 ns/iter reference)
import functools

import jax
import jax.numpy as jnp
from jax import lax
from jax.experimental import pallas as pl
from jax.experimental.pallas import tpu as pltpu

N_DEV = 4
E_PER = 4


def kernel(x, router_W, route_idx, expert_W, shared_W):
    m, d = x.shape
    n_exp, _, h = expert_W.shape

    def body(x_ref, rw_ref, idx_ref, ew_ref, sw_ref, out_ref,
             comm_ref, send_sems, recv_sems):
        my = lax.axis_index("i")
        left = lax.rem(my + N_DEV - 1, N_DEV)
        right = lax.rem(my + 1, N_DEV)

        barrier_sem = pltpu.get_barrier_semaphore()
        for nbr in (left, right):
            pl.semaphore_signal(barrier_sem, inc=1, device_id=(nbr,),
                                device_id_type=pl.DeviceIdType.MESH)
        pl.semaphore_wait(barrier_sem, 2)

        xv = x_ref[:, :]
        idx = idx_ref[:, :]

        scores = jnp.dot(xv, rw_ref[:, :], preferred_element_type=jnp.float32)
        s_max = jnp.max(scores, axis=1, keepdims=True)
        e_s = jnp.exp(scores - s_max)
        probs = e_s / jnp.sum(e_s, axis=1, keepdims=True)
        lanes = lax.broadcasted_iota(jnp.int32, probs.shape, 1)
        gate = jnp.sum(jnp.where(lanes == idx, probs, 0.0),
                       axis=1, keepdims=True)

        acc = jnp.dot(xv, sw_ref[:, :], preferred_element_type=jnp.float32)

        def apply_shard(get_w, owner, acc):
            for j in range(E_PER):
                e = owner * E_PER + j
                coeff = jnp.where(idx == e, gate, 0.0)
                acc = acc + jnp.dot(xv * coeff, get_w(j),
                                    preferred_element_type=jnp.float32)
            return acc

        acc = apply_shard(lambda j: ew_ref[j], my, acc)

        for k in range(N_DEV - 1):
            rdma = pltpu.make_async_remote_copy(
                src_ref=ew_ref if k == 0 else comm_ref.at[k - 1],
                dst_ref=comm_ref.at[k],
                send_sem=send_sems.at[k],
                recv_sem=recv_sems.at[k],
                device_id=(right,),
                device_id_type=pl.DeviceIdType.MESH,
            )
            rdma.start()
            rdma.wait()
            owner = lax.rem(my + N_DEV - 1 - k, N_DEV)
            acc = apply_shard(lambda j: comm_ref[k, j], owner, acc)

        out_ref[:, :] = acc

        @functools.partial(pl.run_scoped,
                           second_barrier=pltpu.SemaphoreType.REGULAR)
        def _(second_barrier):
            for nbr in (left, right):
                pl.semaphore_signal(second_barrier, inc=1, device_id=(nbr,),
                                    device_id_type=pl.DeviceIdType.MESH)
            pl.semaphore_wait(second_barrier, 2)

    return pl.pallas_call(
        body,
        out_shape=jax.ShapeDtypeStruct((m, h), jnp.float32),
        in_specs=[pl.BlockSpec(memory_space=pltpu.VMEM)] * 5,
        out_specs=pl.BlockSpec(memory_space=pltpu.VMEM),
        scratch_shapes=[
            pltpu.VMEM((N_DEV - 1, n_exp, d, h), jnp.float32),
            pltpu.SemaphoreType.DMA((N_DEV - 1,)),
            pltpu.SemaphoreType.DMA((N_DEV - 1,)),
        ],
        compiler_params=pltpu.CompilerParams(collective_id=0),
    )(x, router_W, route_idx, expert_W, shared_W)


# device time: 48202 ns/iter; 1.7847x vs baseline; 1.7847x over previous
import functools

import jax
import jax.numpy as jnp
from jax import lax
from jax.experimental import pallas as pl
from jax.experimental.pallas import tpu as pltpu

N_DEV = 4
E_PER = 4


def kernel(x, router_W, route_idx, expert_W, shared_W):
    m, d = x.shape
    n_exp, _, h = expert_W.shape
    half = n_exp // 2

    def body(x_ref, rw_ref, idx_ref, ew_ref, sw_ref, out_ref,
             rbuf0, lbuf0, rbuf1, lbuf1, send_sems, recv_sems):
        my = lax.axis_index("i")
        left = lax.rem(my + N_DEV - 1, N_DEV)
        right = lax.rem(my + 1, N_DEV)

        barrier_sem = pltpu.get_barrier_semaphore()
        for nbr in (left, right):
            pl.semaphore_signal(barrier_sem, inc=1, device_id=(nbr,),
                                device_id_type=pl.DeviceIdType.MESH)
        pl.semaphore_wait(barrier_sem, 2)

        r0 = pltpu.make_async_remote_copy(
            src_ref=ew_ref, dst_ref=rbuf0,
            send_sem=send_sems.at[0], recv_sem=recv_sems.at[0],
            device_id=(right,), device_id_type=pl.DeviceIdType.MESH)
        r0.start()
        l0 = pltpu.make_async_remote_copy(
            src_ref=ew_ref, dst_ref=lbuf0,
            send_sem=send_sems.at[1], recv_sem=recv_sems.at[1],
            device_id=(left,), device_id_type=pl.DeviceIdType.MESH)
        l0.start()

        xv = x_ref[:, :]
        idx = idx_ref[:, :]

        scores = jnp.dot(xv, rw_ref[:, :], preferred_element_type=jnp.float32)
        s_max = jnp.max(scores, axis=1, keepdims=True)
        e_s = jnp.exp(scores - s_max)
        probs = e_s / jnp.sum(e_s, axis=1, keepdims=True)
        lanes = lax.broadcasted_iota(jnp.int32, probs.shape, 1)
        gate = jnp.sum(jnp.where(lanes == idx, probs, 0.0),
                       axis=1, keepdims=True)

        acc = jnp.dot(xv, sw_ref[:, :], preferred_element_type=jnp.float32)

        def expert_gemm(w, e, acc):
            coeff = jnp.where(idx == e, gate, 0.0)
            return acc + jnp.dot(xv * coeff, w,
                                 preferred_element_type=jnp.float32)

        def apply_shard(get_w, owner, acc, js=range(E_PER)):
            for j in js:
                acc = expert_gemm(get_w(j), owner * E_PER + j, acc)
            return acc

        acc = apply_shard(lambda j: ew_ref[j], my, acc)

        r0.wait()
        r1 = pltpu.make_async_remote_copy(
            src_ref=rbuf0.at[pl.ds(0, half)], dst_ref=rbuf1,
            send_sem=send_sems.at[2], recv_sem=recv_sems.at[2],
            device_id=(right,), device_id_type=pl.DeviceIdType.MESH)
        r1.start()
        l0.wait()
        l1 = pltpu.make_async_remote_copy(
            src_ref=lbuf0.at[pl.ds(half, half)], dst_ref=lbuf1,
            send_sem=send_sems.at[3], recv_sem=recv_sems.at[3],
            device_id=(left,), device_id_type=pl.DeviceIdType.MESH)
        l1.start()

        acc = apply_shard(lambda j: rbuf0[j], left, acc)
        acc = apply_shard(lambda j: lbuf0[j], right, acc)

        r1.wait()
        l1.wait()
        opp = lax.rem(my + 2, N_DEV)
        acc = apply_shard(lambda j: rbuf1[j], opp, acc, js=range(half))
        acc = apply_shard(lambda j: lbuf1[j - half], opp, acc,
                          js=range(half, E_PER))

        out_ref[:, :] = acc

        @functools.partial(pl.run_scoped,
                           second_barrier=pltpu.SemaphoreType.REGULAR)
        def _(second_barrier):
            for nbr in (left, right):
                pl.semaphore_signal(second_barrier, inc=1, device_id=(nbr,),
                                    device_id_type=pl.DeviceIdType.MESH)
            pl.semaphore_wait(second_barrier, 2)

    return pl.pallas_call(
        body,
        out_shape=jax.ShapeDtypeStruct((m, h), jnp.float32),
        in_specs=[pl.BlockSpec(memory_space=pltpu.VMEM)] * 5,
        out_specs=pl.BlockSpec(memory_space=pltpu.VMEM),
        scratch_shapes=[
            pltpu.VMEM((n_exp, d, h), jnp.float32),
            pltpu.VMEM((n_exp, d, h), jnp.float32),
            pltpu.VMEM((half, d, h), jnp.float32),
            pltpu.VMEM((half, d, h), jnp.float32),
            pltpu.SemaphoreType.DMA((4,)),
            pltpu.SemaphoreType.DMA((4,)),
        ],
        compiler_params=pltpu.CompilerParams(collective_id=0),
    )(x, router_W, route_idx, expert_W, shared_W)
